# Initial kernel scaffold; baseline (speedup 1.0000x reference)
import jax
import jax.numpy as jnp
from jax.experimental import pallas as pl


def kernel(node_emb, edge_index, edge_types, edge_attr, W1, q1, k1, We1, e1, b1, g1, be1, W2, q2, k2, We2, e2, b2, g2, be2, Wl, bl):
    raise NotImplementedError("write your pallas kernel here")


# speedup vs baseline: 96.2480x; 96.2480x over previous
import functools
import jax
import jax.numpy as jnp
from jax import lax
from jax.experimental import pallas as pl
from jax.experimental.pallas import tpu as pltpu
from jax.experimental.pallas import tpu_sc as plsc

N = 10000
E = 320000
D = 128
R = 8
HP = 16
EPS = 1e-5

NC = 2
NS = 16
NW = NC * NS
EW = E // NW
C = 2000
NCH = EW // C
NP = 10240
RPT = NP // NS


def _sc_edge_body(xrel, qn, kn, gisrc, gidst, dstn, attr, cvec, out,
                  gisrc_v, gidst_v, dst_v, attr_v, qe_v, ke_v, rows_v,
                  zbuf, cvec_v, accum_sp, sem):
    cid = lax.axis_index("c")
    sid = lax.axis_index("s")
    wid = cid * NS + sid
    base0 = wid * EW

    def zb(i, _):
        zbuf[i] = jnp.zeros((HP,), jnp.float32)
        return 0
    lax.fori_loop(0, RPT, zb, 0)
    pltpu.sync_copy(zbuf, accum_sp.at[pl.ds(sid * RPT, RPT)])
    plsc.subcore_barrier()

    pltpu.sync_copy(cvec, cvec_v)
    cv = cvec_v[...]

    def chunk(k, _):
        base = base0 + k * C
        pltpu.sync_copy(gisrc.at[pl.ds(base, C)], gisrc_v)
        pltpu.sync_copy(gidst.at[pl.ds(base, C)], gidst_v)
        pltpu.sync_copy(dstn.at[pl.ds(base, C)], dst_v)
        pltpu.sync_copy(attr.at[pl.ds(base, C)], attr_v)
        pltpu.async_copy(qn.at[gidst_v], qe_v, sem).wait()
        pltpu.async_copy(kn.at[gisrc_v], ke_v, sem).wait()
        pltpu.async_copy(xrel.at[gisrc_v], rows_v, sem).wait()

        def grp(g, _):
            q16 = qe_v[pl.ds(g * 16, 16)]
            k16 = ke_v[pl.ds(g * 16, 16)]
            a16 = attr_v[pl.ds(g * 16, 16)]
            s = q16 + k16 + a16 * cv
            a = jnp.exp(jnp.maximum(s, 0.2 * s))
            for j in range(16):
                e = g * 16 + j
                rows_v[e] = rows_v[e] * a[j]
            return 0
        lax.fori_loop(0, C // 16, grp, 0)

        pltpu.sync_copy(rows_v, accum_sp.at[dst_v], add=True)
        return 0
    lax.fori_loop(0, NCH, chunk, 0)

    plsc.subcore_barrier()
    off = cid * NP + sid * RPT
    pltpu.sync_copy(accum_sp.at[pl.ds(sid * RPT, RPT)],
                    out.at[pl.ds(off, RPT)])


def _sc_edge(xrel, qn, kn, gisrc, gidst, dstn, attr, cvec):
    mesh = plsc.VectorSubcoreMesh(core_axis_name="c", subcore_axis_name="s")
    f = functools.partial(
        pl.kernel, mesh=mesh,
        compiler_params=pltpu.CompilerParams(use_tc_tiling_on_sc=False),
        out_type=jax.ShapeDtypeStruct((NC * NP, HP), jnp.float32),
        scratch_types=[
            pltpu.VMEM((C,), jnp.int32),
            pltpu.VMEM((C,), jnp.int32),
            pltpu.VMEM((C,), jnp.int32),
            pltpu.VMEM((C,), jnp.float32),
            pltpu.VMEM((C,), jnp.float32),
            pltpu.VMEM((C,), jnp.float32),
            pltpu.VMEM((C, HP), jnp.float32),
            pltpu.VMEM((RPT, HP), jnp.float32),
            pltpu.VMEM((16,), jnp.float32),
            pltpu.VMEM_SHARED((NP, HP), jnp.float32),
            pltpu.SemaphoreType.DMA,
        ],
    )(_sc_edge_body)
    return f(xrel, qn, kn, gisrc, gidst, dstn, attr, cvec)


BN_ROWS = 1000
GRID = N // BN_ROWS


def _ones_lane(x):
    col = lax.broadcasted_iota(jnp.int32, x.shape, 1)
    return jnp.where(col % HP == HP - 1, 1.0, x)


def _d1_body(x_ref, wp_ref, wqk_ref, xrel_ref, qk_ref):
    x = x_ref[...]
    xrel_ref[...] = _ones_lane(
        jnp.dot(x, wp_ref[...], preferred_element_type=jnp.float32))
    qk_ref[...] = jnp.dot(x, wqk_ref[...], preferred_element_type=jnp.float32)


def _dense1(x, wp, wqk):
    return pl.pallas_call(
        _d1_body,
        grid=(GRID,),
        in_specs=[
            pl.BlockSpec((BN_ROWS, D), lambda i: (i, 0)),
            pl.BlockSpec((D, R * HP), lambda i: (0, 0)),
            pl.BlockSpec((D, HP), lambda i: (0, 0)),
        ],
        out_specs=[
            pl.BlockSpec((BN_ROWS, R * HP), lambda i: (i, 0)),
            pl.BlockSpec((BN_ROWS, HP), lambda i: (i, 0)),
        ],
        out_shape=[
            jax.ShapeDtypeStruct((N, R * HP), jnp.float32),
            jax.ShapeDtypeStruct((N, HP), jnp.float32),
        ],
    )(x, wp, wqk)


def _post_act(a0, a1, bv, sv, tv):
    ag = a0 + a1
    denom = ag[:, HP - 1:HP] + 1e-16
    x = (ag / denom + bv) * sv + tv
    return jnp.where(x > 0, x, jnp.exp(jnp.minimum(x, 0.0)) - 1.0)


def _d2_body(a0_ref, a1_ref, wp_ref, wqk_ref, b_ref, s_ref, t_ref,
             xrel_ref, qk_ref):
    act = _post_act(a0_ref[...], a1_ref[...], b_ref[...], s_ref[...],
                    t_ref[...])
    xrel_ref[...] = _ones_lane(
        jnp.dot(act, wp_ref[...], preferred_element_type=jnp.float32))
    qk_ref[...] = jnp.dot(act, wqk_ref[...],
                          preferred_element_type=jnp.float32)


def _dense2(a0, a1, wp, wqk, bv, sv, tv):
    vec = pl.BlockSpec((1, HP), lambda i: (0, 0))
    return pl.pallas_call(
        _d2_body,
        grid=(GRID,),
        in_specs=[
            pl.BlockSpec((BN_ROWS, HP), lambda i: (i, 0)),
            pl.BlockSpec((BN_ROWS, HP), lambda i: (i, 0)),
            pl.BlockSpec((HP, R * HP), lambda i: (0, 0)),
            pl.BlockSpec((HP, HP), lambda i: (0, 0)),
            vec, vec, vec,
        ],
        out_specs=[
            pl.BlockSpec((BN_ROWS, R * HP), lambda i: (i, 0)),
            pl.BlockSpec((BN_ROWS, HP), lambda i: (i, 0)),
        ],
        out_shape=[
            jax.ShapeDtypeStruct((N, R * HP), jnp.float32),
            jax.ShapeDtypeStruct((N, HP), jnp.float32),
        ],
    )(a0, a1, wp, wqk, bv, sv, tv)


def _d3_body(a0_ref, a1_ref, wl_ref, b_ref, s_ref, t_ref, out_ref):
    act = _post_act(a0_ref[...], a1_ref[...], b_ref[...], s_ref[...],
                    t_ref[...])
    out_ref[...] = jnp.dot(act, wl_ref[...],
                           preferred_element_type=jnp.float32)


def _dense3(a0, a1, wl, bv, sv, tv):
    vec = pl.BlockSpec((1, HP), lambda i: (0, 0))
    return pl.pallas_call(
        _d3_body,
        grid=(GRID,),
        in_specs=[
            pl.BlockSpec((BN_ROWS, HP), lambda i: (i, 0)),
            pl.BlockSpec((BN_ROWS, HP), lambda i: (i, 0)),
            pl.BlockSpec((HP, 128), lambda i: (0, 0)),
            vec, vec, vec,
        ],
        out_specs=pl.BlockSpec((BN_ROWS, 128), lambda i: (i, 0)),
        out_shape=jax.ShapeDtypeStruct((N, 128), jnp.float32),
    )(a0, a1, wl, bv, sv, tv)


def _pad16(v, h):
    return jnp.concatenate([v, jnp.zeros((HP - h,), v.dtype)])[None, :]


def kernel(node_emb, edge_index, edge_types, edge_attr, W1, q1, k1, We1, e1,
           b1, g1, be1, W2, q2, k2, We2, e2, b2, g2, be2, Wl, bl):
    H1, H2 = 15, 10
    f32 = jnp.float32

    def fold(W, q, k, h):
        Din = W.shape[1]
        wp = jnp.zeros((Din, R, HP), f32).at[:, :, :h].set(
            jnp.transpose(W, (1, 0, 2))).reshape(Din, R * HP)
        wq = jnp.einsum('rih,ho->ir', W, q)
        wk = jnp.einsum('rih,ho->ir', W, k)
        return wp, jnp.concatenate([wq, wk], axis=1)

    wp1, wqk1 = fold(W1, q1, k1, H1)
    c1 = jnp.full((16,), (We1 @ e1)[0, 0], f32)
    c2 = jnp.full((16,), (We2 @ e2)[0, 0], f32)

    wp2_in = jnp.zeros((HP, R, HP), f32).at[:H1, :, :H2].set(
        jnp.transpose(W2, (1, 0, 2))).reshape(HP, R * HP)
    wq2 = jnp.zeros((HP, R), f32).at[:H1].set(jnp.einsum('rih,ho->ir', W2, q2))
    wk2 = jnp.zeros((HP, R), f32).at[:H1].set(jnp.einsum('rih,ho->ir', W2, k2))
    wqk2 = jnp.concatenate([wq2, wk2], axis=1)

    wl16 = jnp.zeros((HP, 128), f32).at[:H2, 0].set(Wl[:, 0])

    b1p = _pad16(b1, H1)
    s1p = _pad16(g1 / jnp.sqrt(1.0 + EPS), H1)
    t1p = _pad16(be1, H1)
    b2p = _pad16(b2, H2)
    s2p = _pad16(g2 / jnp.sqrt(1.0 + EPS), H2)
    t2p = _pad16(be2, H2)

    src, dst = edge_index[0], edge_index[1]
    gisrc = src * R + edge_types
    gidst = dst * R + edge_types
    attr = edge_attr[:, 0]

    xrel1, qk1 = _dense1(node_emb, wp1, wqk1)
    ag1 = _sc_edge(xrel1.reshape(N * R, HP),
                   qk1[:, :R].reshape(-1), qk1[:, R:].reshape(-1),
                   gisrc, gidst, dst, attr, c1)
    a10, a11 = ag1[:N], ag1[NP:NP + N]

    xrel2, qk2 = _dense2(a10, a11, wp2_in, wqk2, b1p, s1p, t1p)
    ag2 = _sc_edge(xrel2.reshape(N * R, HP),
                   qk2[:, :R].reshape(-1), qk2[:, R:].reshape(-1),
                   gisrc, gidst, dst, attr, c2)

    out = _dense3(ag2[:N], ag2[NP:NP + N], wl16, b2p, s2p, t2p)
    return out[:, :1] + bl


# speedup vs baseline: 102.8649x; 1.0687x over previous
import functools
import jax
import jax.numpy as jnp
from jax import lax
from jax.experimental import pallas as pl
from jax.experimental.pallas import tpu as pltpu
from jax.experimental.pallas import tpu_sc as plsc

N = 10000
E = 320000
D = 128
R = 8
HP = 16
EPS = 1e-5

NC = 2
NS = 16
NW = NC * NS
EW = E // NW
C = 2000
NCH = EW // C
NP = 10240
RPT = NP // NS


def _sc_edge_body(xrel, qn, kn, gisrc, gidst, dstn, attr, cvec, out,
                  gisrc_v, gidst_v, dst_v, attr_v, qe_v, ke_v, rows_v,
                  zbuf, cvec_v, accum_sp, sem, sem2, sem3):
    cid = lax.axis_index("c")
    sid = lax.axis_index("s")
    wid = cid * NS + sid
    base0 = wid * EW

    def zb(i, _):
        zbuf[i] = jnp.zeros((HP,), jnp.float32)
        return 0
    lax.fori_loop(0, RPT, zb, 0)
    pltpu.sync_copy(zbuf, accum_sp.at[pl.ds(sid * RPT, RPT)])
    plsc.subcore_barrier()

    pltpu.sync_copy(cvec, cvec_v)
    cv = cvec_v[...]

    def chunk(k, _):
        base = base0 + k * C
        pltpu.sync_copy(gisrc.at[pl.ds(base, C)], gisrc_v)
        pltpu.sync_copy(gidst.at[pl.ds(base, C)], gidst_v)
        pltpu.sync_copy(dstn.at[pl.ds(base, C)], dst_v)
        pltpu.sync_copy(attr.at[pl.ds(base, C)], attr_v)
        cp1 = pltpu.async_copy(qn.at[gidst_v], qe_v, sem)
        cp2 = pltpu.async_copy(kn.at[gisrc_v], ke_v, sem2)
        cp3 = pltpu.async_copy(xrel.at[gisrc_v], rows_v, sem3)
        cp1.wait()
        cp2.wait()
        cp3.wait()

        def grp(g, _):
            q16 = qe_v[pl.ds(g * 16, 16)]
            k16 = ke_v[pl.ds(g * 16, 16)]
            a16 = attr_v[pl.ds(g * 16, 16)]
            s = q16 + k16 + a16 * cv
            a = jnp.exp(jnp.maximum(s, 0.2 * s))
            for j in range(16):
                e = g * 16 + j
                rows_v[e] = rows_v[e] * a[j]
            return 0
        lax.fori_loop(0, C // 16, grp, 0)

        pltpu.sync_copy(rows_v, accum_sp.at[dst_v], add=True)
        return 0
    lax.fori_loop(0, NCH, chunk, 0)

    plsc.subcore_barrier()
    off = cid * NP + sid * RPT
    pltpu.sync_copy(accum_sp.at[pl.ds(sid * RPT, RPT)],
                    out.at[pl.ds(off, RPT)])


def _sc_edge(xrel, qn, kn, gisrc, gidst, dstn, attr, cvec):
    mesh = plsc.VectorSubcoreMesh(core_axis_name="c", subcore_axis_name="s")
    f = functools.partial(
        pl.kernel, mesh=mesh,
        compiler_params=pltpu.CompilerParams(use_tc_tiling_on_sc=False),
        out_type=jax.ShapeDtypeStruct((NC * NP, HP), jnp.float32),
        scratch_types=[
            pltpu.VMEM((C,), jnp.int32),
            pltpu.VMEM((C,), jnp.int32),
            pltpu.VMEM((C,), jnp.int32),
            pltpu.VMEM((C,), jnp.float32),
            pltpu.VMEM((C,), jnp.float32),
            pltpu.VMEM((C,), jnp.float32),
            pltpu.VMEM((C, HP), jnp.float32),
            pltpu.VMEM((RPT, HP), jnp.float32),
            pltpu.VMEM((16,), jnp.float32),
            pltpu.VMEM_SHARED((NP, HP), jnp.float32),
            pltpu.SemaphoreType.DMA,
            pltpu.SemaphoreType.DMA,
            pltpu.SemaphoreType.DMA,
        ],
    )(_sc_edge_body)
    return f(xrel, qn, kn, gisrc, gidst, dstn, attr, cvec)


BN_ROWS = 1000
GRID = N // BN_ROWS


def _ones_lane(x):
    col = lax.broadcasted_iota(jnp.int32, x.shape, 1)
    return jnp.where(col % HP == HP - 1, 1.0, x)


def _d1_body(x_ref, wp_ref, wqk_ref, xrel_ref, qk_ref):
    x = x_ref[...]
    xrel_ref[...] = _ones_lane(
        jnp.dot(x, wp_ref[...], preferred_element_type=jnp.float32))
    qk_ref[...] = jnp.dot(x, wqk_ref[...], preferred_element_type=jnp.float32)


def _dense1(x, wp, wqk):
    return pl.pallas_call(
        _d1_body,
        grid=(GRID,),
        in_specs=[
            pl.BlockSpec((BN_ROWS, D), lambda i: (i, 0)),
            pl.BlockSpec((D, R * HP), lambda i: (0, 0)),
            pl.BlockSpec((D, HP), lambda i: (0, 0)),
        ],
        out_specs=[
            pl.BlockSpec((BN_ROWS, R * HP), lambda i: (i, 0)),
            pl.BlockSpec((BN_ROWS, HP), lambda i: (i, 0)),
        ],
        out_shape=[
            jax.ShapeDtypeStruct((N, R * HP), jnp.float32),
            jax.ShapeDtypeStruct((N, HP), jnp.float32),
        ],
    )(x, wp, wqk)


def _post_act(a0, a1, bv, sv, tv):
    ag = a0 + a1
    denom = ag[:, HP - 1:HP] + 1e-16
    x = (ag / denom + bv) * sv + tv
    return jnp.where(x > 0, x, jnp.exp(jnp.minimum(x, 0.0)) - 1.0)


def _d2_body(a0_ref, a1_ref, wp_ref, wqk_ref, b_ref, s_ref, t_ref,
             xrel_ref, qk_ref):
    act = _post_act(a0_ref[...], a1_ref[...], b_ref[...], s_ref[...],
                    t_ref[...])
    xrel_ref[...] = _ones_lane(
        jnp.dot(act, wp_ref[...], preferred_element_type=jnp.float32))
    qk_ref[...] = jnp.dot(act, wqk_ref[...],
                          preferred_element_type=jnp.float32)


def _dense2(a0, a1, wp, wqk, bv, sv, tv):
    vec = pl.BlockSpec((1, HP), lambda i: (0, 0))
    return pl.pallas_call(
        _d2_body,
        grid=(GRID,),
        in_specs=[
            pl.BlockSpec((BN_ROWS, HP), lambda i: (i, 0)),
            pl.BlockSpec((BN_ROWS, HP), lambda i: (i, 0)),
            pl.BlockSpec((HP, R * HP), lambda i: (0, 0)),
            pl.BlockSpec((HP, HP), lambda i: (0, 0)),
            vec, vec, vec,
        ],
        out_specs=[
            pl.BlockSpec((BN_ROWS, R * HP), lambda i: (i, 0)),
            pl.BlockSpec((BN_ROWS, HP), lambda i: (i, 0)),
        ],
        out_shape=[
            jax.ShapeDtypeStruct((N, R * HP), jnp.float32),
            jax.ShapeDtypeStruct((N, HP), jnp.float32),
        ],
    )(a0, a1, wp, wqk, bv, sv, tv)


def _d3_body(a0_ref, a1_ref, wl_ref, b_ref, s_ref, t_ref, out_ref):
    act = _post_act(a0_ref[...], a1_ref[...], b_ref[...], s_ref[...],
                    t_ref[...])
    out_ref[...] = jnp.dot(act, wl_ref[...],
                           preferred_element_type=jnp.float32)


def _dense3(a0, a1, wl, bv, sv, tv):
    vec = pl.BlockSpec((1, HP), lambda i: (0, 0))
    return pl.pallas_call(
        _d3_body,
        grid=(GRID,),
        in_specs=[
            pl.BlockSpec((BN_ROWS, HP), lambda i: (i, 0)),
            pl.BlockSpec((BN_ROWS, HP), lambda i: (i, 0)),
            pl.BlockSpec((HP, 128), lambda i: (0, 0)),
            vec, vec, vec,
        ],
        out_specs=pl.BlockSpec((BN_ROWS, 128), lambda i: (i, 0)),
        out_shape=jax.ShapeDtypeStruct((N, 128), jnp.float32),
    )(a0, a1, wl, bv, sv, tv)


def _pad16(v, h):
    return jnp.concatenate([v, jnp.zeros((HP - h,), v.dtype)])[None, :]


def kernel(node_emb, edge_index, edge_types, edge_attr, W1, q1, k1, We1, e1,
           b1, g1, be1, W2, q2, k2, We2, e2, b2, g2, be2, Wl, bl):
    H1, H2 = 15, 10
    f32 = jnp.float32

    def fold(W, q, k, h):
        Din = W.shape[1]
        wp = jnp.zeros((Din, R, HP), f32).at[:, :, :h].set(
            jnp.transpose(W, (1, 0, 2))).reshape(Din, R * HP)
        wq = jnp.einsum('rih,ho->ir', W, q)
        wk = jnp.einsum('rih,ho->ir', W, k)
        return wp, jnp.concatenate([wq, wk], axis=1)

    wp1, wqk1 = fold(W1, q1, k1, H1)
    c1 = jnp.full((16,), (We1 @ e1)[0, 0], f32)
    c2 = jnp.full((16,), (We2 @ e2)[0, 0], f32)

    wp2_in = jnp.zeros((HP, R, HP), f32).at[:H1, :, :H2].set(
        jnp.transpose(W2, (1, 0, 2))).reshape(HP, R * HP)
    wq2 = jnp.zeros((HP, R), f32).at[:H1].set(jnp.einsum('rih,ho->ir', W2, q2))
    wk2 = jnp.zeros((HP, R), f32).at[:H1].set(jnp.einsum('rih,ho->ir', W2, k2))
    wqk2 = jnp.concatenate([wq2, wk2], axis=1)

    wl16 = jnp.zeros((HP, 128), f32).at[:H2, 0].set(Wl[:, 0])

    b1p = _pad16(b1, H1)
    s1p = _pad16(g1 / jnp.sqrt(1.0 + EPS), H1)
    t1p = _pad16(be1, H1)
    b2p = _pad16(b2, H2)
    s2p = _pad16(g2 / jnp.sqrt(1.0 + EPS), H2)
    t2p = _pad16(be2, H2)

    src, dst = edge_index[0], edge_index[1]
    gisrc = src * R + edge_types
    gidst = dst * R + edge_types
    attr = edge_attr[:, 0]

    xrel1, qk1 = _dense1(node_emb, wp1, wqk1)
    ag1 = _sc_edge(xrel1.reshape(N * R, HP),
                   qk1[:, :R].reshape(-1), qk1[:, R:].reshape(-1),
                   gisrc, gidst, dst, attr, c1)
    a10, a11 = ag1[:N], ag1[NP:NP + N]

    xrel2, qk2 = _dense2(a10, a11, wp2_in, wqk2, b1p, s1p, t1p)
    ag2 = _sc_edge(xrel2.reshape(N * R, HP),
                   qk2[:, :R].reshape(-1), qk2[:, R:].reshape(-1),
                   gisrc, gidst, dst, attr, c2)

    out = _dense3(ag2[:N], ag2[NP:NP + N], wl16, b2p, s2p, t2p)
    return out[:, :1] + bl


# speedup vs baseline: 123.4831x; 1.2004x over previous
import functools
import jax
import jax.numpy as jnp
from jax import lax
from jax.experimental import pallas as pl
from jax.experimental.pallas import tpu as pltpu
from jax.experimental.pallas import tpu_sc as plsc

N = 10000
E = 320000
D = 128
R = 8
HP = 16
EPS = 1e-5

NC = 2
NS = 16
NW = NC * NS
EW = E // NW
C = 2000
NCH = EW // C
NP = 10240
RPT = NP // NS


def _sc_edge_body(xrel, qn, kn, gisrc, gidst, dstn, attr, cvec, out,
                  gisrc_v, gidst_v, dst_v, attr_v, qe_v, ke_v, rows_v,
                  zbuf, cvec_v, accum_sp, lsem, gsem, ssem):
    cid = lax.axis_index("c")
    sid = lax.axis_index("s")
    wid = cid * NS + sid
    base0 = wid * EW

    def zb(i, _):
        zbuf[i] = jnp.zeros((HP,), jnp.float32)
        return 0
    lax.fori_loop(0, RPT, zb, 0)
    pltpu.sync_copy(zbuf, accum_sp.at[pl.ds(sid * RPT, RPT)])
    plsc.subcore_barrier()

    pltpu.sync_copy(cvec, cvec_v)
    cv = cvec_v[...]

    def fire_linear(b, k):
        base = base0 + k * C
        return [
            pltpu.async_copy(gisrc.at[pl.ds(base, C)], gisrc_v.at[b],
                             lsem.at[b]),
            pltpu.async_copy(gidst.at[pl.ds(base, C)], gidst_v.at[b],
                             lsem.at[b]),
            pltpu.async_copy(dstn.at[pl.ds(base, C)], dst_v.at[b],
                             lsem.at[b]),
            pltpu.async_copy(attr.at[pl.ds(base, C)], attr_v.at[b],
                             lsem.at[b]),
        ]

    def fire_gathers(b):
        return [
            pltpu.async_copy(qn.at[gidst_v.at[b]], qe_v.at[b], gsem.at[b, 0]),
            pltpu.async_copy(kn.at[gisrc_v.at[b]], ke_v.at[b], gsem.at[b, 1]),
            pltpu.async_copy(xrel.at[gisrc_v.at[b]], rows_v.at[b],
                             gsem.at[b, 2]),
        ]

    lin = {0: fire_linear(0, 0)}
    for cp in lin[0]:
        cp.wait()
    gth = {0: fire_gathers(0)}
    if NCH > 1:
        lin[1] = fire_linear(1, 1)
    for k in range(NCH):
        b = k % 2
        for cp in gth[k]:
            cp.wait()
        if k + 1 < NCH:
            for cp in lin[k + 1]:
                cp.wait()
            gth[k + 1] = fire_gathers(1 - b)

        def grp(g, _):
            q16 = qe_v[b, pl.ds(g * 16, 16)]
            k16 = ke_v[b, pl.ds(g * 16, 16)]
            a16 = attr_v[b, pl.ds(g * 16, 16)]
            s = q16 + k16 + a16 * cv
            a = jnp.exp(jnp.maximum(s, 0.2 * s))
            for j in range(16):
                e = g * 16 + j
                rows_v[b, e] = rows_v[b, e] * a[j]
            return 0
        lax.fori_loop(0, C // 16, grp, 0)

        pltpu.async_copy(rows_v.at[b], accum_sp.at[dst_v.at[b]], ssem.at[b],
                         add=True).wait()
        if k + 2 < NCH:
            lin[k + 2] = fire_linear(b, k + 2)

    plsc.subcore_barrier()
    off = cid * NP + sid * RPT
    pltpu.sync_copy(accum_sp.at[pl.ds(sid * RPT, RPT)],
                    out.at[pl.ds(off, RPT)])


def _sc_edge(xrel, qn, kn, gisrc, gidst, dstn, attr, cvec):
    mesh = plsc.VectorSubcoreMesh(core_axis_name="c", subcore_axis_name="s")
    f = functools.partial(
        pl.kernel, mesh=mesh,
        compiler_params=pltpu.CompilerParams(use_tc_tiling_on_sc=False),
        out_type=jax.ShapeDtypeStruct((NC * NP, HP), jnp.float32),
        scratch_types=[
            pltpu.VMEM((2, C), jnp.int32),
            pltpu.VMEM((2, C), jnp.int32),
            pltpu.VMEM((2, C), jnp.int32),
            pltpu.VMEM((2, C), jnp.float32),
            pltpu.VMEM((2, C), jnp.float32),
            pltpu.VMEM((2, C), jnp.float32),
            pltpu.VMEM((2, C, HP), jnp.float32),
            pltpu.VMEM((RPT, HP), jnp.float32),
            pltpu.VMEM((16,), jnp.float32),
            pltpu.VMEM_SHARED((NP, HP), jnp.float32),
            pltpu.SemaphoreType.DMA((2,)),
            pltpu.SemaphoreType.DMA((2, 3)),
            pltpu.SemaphoreType.DMA((2,)),
        ],
    )(_sc_edge_body)
    return f(xrel, qn, kn, gisrc, gidst, dstn, attr, cvec)


BN_ROWS = 1000
GRID = N // BN_ROWS


def _ones_lane(x):
    col = lax.broadcasted_iota(jnp.int32, x.shape, 1)
    return jnp.where(col % HP == HP - 1, 1.0, x)


def _d1_body(x_ref, wp_ref, wqk_ref, xrel_ref, qk_ref):
    x = x_ref[...]
    xrel_ref[...] = _ones_lane(
        jnp.dot(x, wp_ref[...], preferred_element_type=jnp.float32))
    qk_ref[...] = jnp.dot(x, wqk_ref[...], preferred_element_type=jnp.float32)


def _dense1(x, wp, wqk):
    return pl.pallas_call(
        _d1_body,
        grid=(GRID,),
        in_specs=[
            pl.BlockSpec((BN_ROWS, D), lambda i: (i, 0)),
            pl.BlockSpec((D, R * HP), lambda i: (0, 0)),
            pl.BlockSpec((D, HP), lambda i: (0, 0)),
        ],
        out_specs=[
            pl.BlockSpec((BN_ROWS, R * HP), lambda i: (i, 0)),
            pl.BlockSpec((BN_ROWS, HP), lambda i: (i, 0)),
        ],
        out_shape=[
            jax.ShapeDtypeStruct((N, R * HP), jnp.float32),
            jax.ShapeDtypeStruct((N, HP), jnp.float32),
        ],
    )(x, wp, wqk)


def _post_act(a0, a1, bv, sv, tv):
    ag = a0 + a1
    denom = ag[:, HP - 1:HP] + 1e-16
    x = (ag / denom + bv) * sv + tv
    return jnp.where(x > 0, x, jnp.exp(jnp.minimum(x, 0.0)) - 1.0)


def _d2_body(a0_ref, a1_ref, wp_ref, wqk_ref, b_ref, s_ref, t_ref,
             xrel_ref, qk_ref):
    act = _post_act(a0_ref[...], a1_ref[...], b_ref[...], s_ref[...],
                    t_ref[...])
    xrel_ref[...] = _ones_lane(
        jnp.dot(act, wp_ref[...], preferred_element_type=jnp.float32))
    qk_ref[...] = jnp.dot(act, wqk_ref[...],
                          preferred_element_type=jnp.float32)


def _dense2(a0, a1, wp, wqk, bv, sv, tv):
    vec = pl.BlockSpec((1, HP), lambda i: (0, 0))
    return pl.pallas_call(
        _d2_body,
        grid=(GRID,),
        in_specs=[
            pl.BlockSpec((BN_ROWS, HP), lambda i: (i, 0)),
            pl.BlockSpec((BN_ROWS, HP), lambda i: (i, 0)),
            pl.BlockSpec((HP, R * HP), lambda i: (0, 0)),
            pl.BlockSpec((HP, HP), lambda i: (0, 0)),
            vec, vec, vec,
        ],
        out_specs=[
            pl.BlockSpec((BN_ROWS, R * HP), lambda i: (i, 0)),
            pl.BlockSpec((BN_ROWS, HP), lambda i: (i, 0)),
        ],
        out_shape=[
            jax.ShapeDtypeStruct((N, R * HP), jnp.float32),
            jax.ShapeDtypeStruct((N, HP), jnp.float32),
        ],
    )(a0, a1, wp, wqk, bv, sv, tv)


def _d3_body(a0_ref, a1_ref, wl_ref, b_ref, s_ref, t_ref, out_ref):
    act = _post_act(a0_ref[...], a1_ref[...], b_ref[...], s_ref[...],
                    t_ref[...])
    out_ref[...] = jnp.dot(act, wl_ref[...],
                           preferred_element_type=jnp.float32)


def _dense3(a0, a1, wl, bv, sv, tv):
    vec = pl.BlockSpec((1, HP), lambda i: (0, 0))
    return pl.pallas_call(
        _d3_body,
        grid=(GRID,),
        in_specs=[
            pl.BlockSpec((BN_ROWS, HP), lambda i: (i, 0)),
            pl.BlockSpec((BN_ROWS, HP), lambda i: (i, 0)),
            pl.BlockSpec((HP, 128), lambda i: (0, 0)),
            vec, vec, vec,
        ],
        out_specs=pl.BlockSpec((BN_ROWS, 128), lambda i: (i, 0)),
        out_shape=jax.ShapeDtypeStruct((N, 128), jnp.float32),
    )(a0, a1, wl, bv, sv, tv)


def _pad16(v, h):
    return jnp.concatenate([v, jnp.zeros((HP - h,), v.dtype)])[None, :]


def kernel(node_emb, edge_index, edge_types, edge_attr, W1, q1, k1, We1, e1,
           b1, g1, be1, W2, q2, k2, We2, e2, b2, g2, be2, Wl, bl):
    H1, H2 = 15, 10
    f32 = jnp.float32

    def fold(W, q, k, h):
        Din = W.shape[1]
        wp = jnp.zeros((Din, R, HP), f32).at[:, :, :h].set(
            jnp.transpose(W, (1, 0, 2))).reshape(Din, R * HP)
        wq = jnp.einsum('rih,ho->ir', W, q)
        wk = jnp.einsum('rih,ho->ir', W, k)
        return wp, jnp.concatenate([wq, wk], axis=1)

    wp1, wqk1 = fold(W1, q1, k1, H1)
    c1 = jnp.full((16,), (We1 @ e1)[0, 0], f32)
    c2 = jnp.full((16,), (We2 @ e2)[0, 0], f32)

    wp2_in = jnp.zeros((HP, R, HP), f32).at[:H1, :, :H2].set(
        jnp.transpose(W2, (1, 0, 2))).reshape(HP, R * HP)
    wq2 = jnp.zeros((HP, R), f32).at[:H1].set(jnp.einsum('rih,ho->ir', W2, q2))
    wk2 = jnp.zeros((HP, R), f32).at[:H1].set(jnp.einsum('rih,ho->ir', W2, k2))
    wqk2 = jnp.concatenate([wq2, wk2], axis=1)

    wl16 = jnp.zeros((HP, 128), f32).at[:H2, 0].set(Wl[:, 0])

    b1p = _pad16(b1, H1)
    s1p = _pad16(g1 / jnp.sqrt(1.0 + EPS), H1)
    t1p = _pad16(be1, H1)
    b2p = _pad16(b2, H2)
    s2p = _pad16(g2 / jnp.sqrt(1.0 + EPS), H2)
    t2p = _pad16(be2, H2)

    src, dst = edge_index[0], edge_index[1]
    gisrc = src * R + edge_types
    gidst = dst * R + edge_types
    attr = edge_attr[:, 0]

    xrel1, qk1 = _dense1(node_emb, wp1, wqk1)
    ag1 = _sc_edge(xrel1.reshape(N * R, HP),
                   qk1[:, :R].reshape(-1), qk1[:, R:].reshape(-1),
                   gisrc, gidst, dst, attr, c1)
    a10, a11 = ag1[:N], ag1[NP:NP + N]

    xrel2, qk2 = _dense2(a10, a11, wp2_in, wqk2, b1p, s1p, t1p)
    ag2 = _sc_edge(xrel2.reshape(N * R, HP),
                   qk2[:, :R].reshape(-1), qk2[:, R:].reshape(-1),
                   gisrc, gidst, dst, attr, c2)

    out = _dense3(ag2[:N], ag2[NP:NP + N], wl16, b2p, s2p, t2p)
    return out[:, :1] + bl
